# baseline (speedup 1.0000x reference)
import functools

import jax
import jax.numpy as jnp
from jax import lax
from jax.experimental import pallas as pl
from jax.experimental.pallas import tpu as pltpu
from jax.experimental.pallas import tpu_sc as plsc

VOCAB = 100000
HIDDEN = 64
BATCH = 1024

_NUM_CORES = 2
_NUM_SUBCORES = 16
_NW = _NUM_CORES * _NUM_SUBCORES
_H_PER_TEC = HIDDEN // _NW
_ELEMS = _H_PER_TEC * BATCH
_N_GATHER = _ELEMS // 128

V_TILE = 2560
N_V = -(-VOCAB // V_TILE)


def _gather_body(tab_hbm, ids_hbm, out_hbm, ids_v, idx_v, rows_v, sem):
    wid = lax.axis_index("s") * _NUM_CORES + lax.axis_index("c")
    pltpu.sync_copy(ids_hbm, ids_v)
    h0 = wid * _H_PER_TEC
    for j in range(_ELEMS // 16):
        h = j // (BATCH // 16)
        i = j % (BATCH // 16)
        idx_v[pl.ds(j * 16, 16)] = ids_v[pl.ds(i * 16, 16)] + (h0 + h) * VOCAB
    copies = [
        pltpu.async_copy(
            tab_hbm.at[idx_v.at[pl.ds(g * 128, 128)]],
            rows_v.at[pl.ds(g * 128, 128)],
            sem,
        )
        for g in range(_N_GATHER)
    ]
    for c in copies:
        c.wait()
    pltpu.sync_copy(rows_v, out_hbm.at[pl.ds(wid * _ELEMS, _ELEMS)])


@functools.cache
def _sc_gather():
    return pl.kernel(
        _gather_body,
        out_type=jax.ShapeDtypeStruct((HIDDEN * BATCH,), jnp.float32),
        mesh=plsc.VectorSubcoreMesh(core_axis_name="c", subcore_axis_name="s"),
        scratch_types=[
            pltpu.VMEM((BATCH,), jnp.int32),
            pltpu.VMEM((_ELEMS,), jnp.int32),
            pltpu.VMEM((_ELEMS,), jnp.float32),
            pltpu.SemaphoreType.DMA,
        ],
        compiler_params=pltpu.CompilerParams(use_tc_tiling_on_sc=False),
    )


def _matmul_body(wt_ref, xt_ref, b_ref, out_ref):
    acc = lax.dot_general(
        wt_ref[...], xt_ref[...],
        dimension_numbers=(((0,), (0,)), ((), ())),
        preferred_element_type=jnp.float32,
    )
    out_ref[...] = acc + b_ref[...].T


def kernel(input_ids, emb_table, W, b):
    ids = input_ids.astype(jnp.int32)
    one = 1.0 - b[0] * 0.0
    tab = (emb_table.T * one).reshape(-1)
    xt = _sc_gather()(tab, ids).reshape(HIDDEN, BATCH)
    wt = W.T
    b2 = jnp.pad(b, (0, N_V * V_TILE - VOCAB)).reshape(1, N_V * V_TILE)
    out_t = pl.pallas_call(
        _matmul_body,
        grid=(N_V,),
        in_specs=[
            pl.BlockSpec((HIDDEN, V_TILE), lambda v: (0, v)),
            pl.BlockSpec((HIDDEN, BATCH), lambda v: (0, 0)),
            pl.BlockSpec((1, V_TILE), lambda v: (0, v)),
        ],
        out_specs=pl.BlockSpec((V_TILE, BATCH), lambda v: (v, 0)),
        out_shape=jax.ShapeDtypeStruct((VOCAB, BATCH), jnp.float32),
        compiler_params=pltpu.CompilerParams(
            dimension_semantics=("arbitrary",),
            vmem_limit_bytes=63 * 1024 * 1024,
        ),
    )(wt, xt, b2)
    return out_t.T


# speedup vs baseline: 1.1075x; 1.1075x over previous
import functools

import jax
import jax.numpy as jnp
from jax import lax
from jax.experimental import pallas as pl
from jax.experimental.pallas import tpu as pltpu
from jax.experimental.pallas import tpu_sc as plsc

VOCAB = 100000
HIDDEN = 64
BATCH = 1024

_NUM_CORES = 2
_NUM_SUBCORES = 16
_NW = _NUM_CORES * _NUM_SUBCORES
_H_PER_TEC = HIDDEN // _NW
_ELEMS = _H_PER_TEC * BATCH
_N_GATHER = _ELEMS // 128

V_TILE = 3200
N_V = -(-VOCAB // V_TILE)


def _gather_body(tab_hbm, ids_hbm, out_hbm, ids_v, idx_v, rows_v, sem):
    wid = lax.axis_index("s") * _NUM_CORES + lax.axis_index("c")
    pltpu.sync_copy(ids_hbm, ids_v)
    h0 = wid * _H_PER_TEC
    for j in range(_ELEMS // 16):
        h = j // (BATCH // 16)
        i = j % (BATCH // 16)
        idx_v[pl.ds(j * 16, 16)] = ids_v[pl.ds(i * 16, 16)] + (h0 + h) * VOCAB
    copies = [
        pltpu.async_copy(
            tab_hbm.at[idx_v.at[pl.ds(g * 128, 128)]],
            rows_v.at[pl.ds(g * 128, 128)],
            sem,
        )
        for g in range(_N_GATHER)
    ]
    for c in copies:
        c.wait()
    pltpu.sync_copy(rows_v, out_hbm.at[pl.ds(wid * _ELEMS, _ELEMS)])


@functools.cache
def _sc_gather():
    return pl.kernel(
        _gather_body,
        out_type=jax.ShapeDtypeStruct((HIDDEN * BATCH,), jnp.float32),
        mesh=plsc.VectorSubcoreMesh(core_axis_name="c", subcore_axis_name="s"),
        scratch_types=[
            pltpu.VMEM((BATCH,), jnp.int32),
            pltpu.VMEM((_ELEMS,), jnp.int32),
            pltpu.VMEM((_ELEMS,), jnp.float32),
            pltpu.SemaphoreType.DMA,
        ],
        compiler_params=pltpu.CompilerParams(use_tc_tiling_on_sc=False),
    )


def _matmul_body(wt_ref, xt_ref, b_ref, out_ref):
    acc = lax.dot_general(
        wt_ref[...], xt_ref[...],
        dimension_numbers=(((0,), (0,)), ((), ())),
        preferred_element_type=jnp.float32,
    )
    out_ref[...] = acc + b_ref[...].T


def kernel(input_ids, emb_table, W, b):
    ids = input_ids.astype(jnp.int32)
    tab = emb_table.T.reshape(-1)
    xt = _sc_gather()(tab, ids).reshape(HIDDEN, BATCH)
    wt = W.T
    b2 = jnp.pad(b, (0, N_V * V_TILE - VOCAB)).reshape(1, N_V * V_TILE)
    out_t = pl.pallas_call(
        _matmul_body,
        grid=(N_V,),
        in_specs=[
            pl.BlockSpec((HIDDEN, V_TILE), lambda v: (0, v)),
            pl.BlockSpec((HIDDEN, BATCH), lambda v: (0, 0)),
            pl.BlockSpec((1, V_TILE), lambda v: (0, v)),
        ],
        out_specs=pl.BlockSpec((V_TILE, BATCH), lambda v: (v, 0)),
        out_shape=jax.ShapeDtypeStruct((VOCAB, BATCH), jnp.float32),
        compiler_params=pltpu.CompilerParams(
            dimension_semantics=("arbitrary",),
            vmem_limit_bytes=63 * 1024 * 1024,
        ),
    )(wt, xt, b2)
    return out_t.T
